# baseline (speedup 1.0000x reference)
import functools

import jax
import jax.numpy as jnp
from jax import lax
from jax.experimental import pallas as pl
from jax.experimental.pallas import tpu as pltpu
from jax.experimental.pallas import tpu_sc as plsc


def _make_deg_kernel(E, N_PAD, K, NC, NS):
    EW = E // (NC * NS)
    NCHUNK = EW // K
    RPT = N_PAD // NS
    mesh = plsc.VectorSubcoreMesh(core_axis_name="c", subcore_axis_name="s")

    BUF = ((RPT + 15) // 16) * 16
    GRP = 5
    assert NCHUNK % GRP == 0

    @functools.partial(
        pl.kernel,
        mesh=mesh,
        compiler_params=pltpu.CompilerParams(use_tc_tiling_on_sc=False),
        out_type=jax.ShapeDtypeStruct((NC * N_PAD,), jnp.float32),
        scratch_types=[
            pltpu.VMEM((NCHUNK, K), jnp.int32),
            pltpu.VMEM((K,), jnp.float32),
            pltpu.VMEM((BUF,), jnp.float32),
            pltpu.VMEM_SHARED((N_PAD,), jnp.float32),
            pltpu.SemaphoreType.DMA,
        ],
    )
    def deg_kernel(dst_hbm, out_hbm, dstb, onesv, buf, acc, sem):
        c = lax.axis_index("c")
        s = lax.axis_index("s")
        w = c * NS + s
        for i in range(K // 16):
            onesv[pl.ds(i * 16, 16)] = jnp.ones((16,), jnp.float32)

        def zstep(i, carry):
            buf[pl.ds(i * 16, 16)] = jnp.zeros((16,), jnp.float32)
            return carry

        lax.fori_loop(0, BUF // 16, zstep, 0, unroll=False)
        pltpu.sync_copy(dst_hbm.at[pl.ds(w * NCHUNK, NCHUNK)], dstb)
        pltpu.sync_copy(buf.at[pl.ds(0, RPT)], acc.at[pl.ds(s * RPT, RPT)])
        plsc.subcore_barrier()

        def step(g, carry):
            for i in range(GRP):
                pltpu.async_copy(onesv, acc.at[dstb.at[g * GRP + i]], sem,
                                 add=True)
            for i in range(GRP):
                pltpu.make_async_copy(onesv, acc.at[dstb.at[g * GRP + i]],
                                      sem).wait()
            return carry

        lax.fori_loop(0, NCHUNK // GRP, step, 0, unroll=False)
        plsc.subcore_barrier()
        pltpu.sync_copy(acc.at[pl.ds(s * RPT, RPT)], buf.at[pl.ds(0, RPT)])
        pltpu.sync_copy(buf.at[pl.ds(0, RPT)],
                        out_hbm.at[pl.ds(c * N_PAD + s * RPT, RPT)])

    return deg_kernel


def _make_scatter_kernel(N, E, N_PAD, D, K, NC, NS):
    EW = E // (NC * NS)
    NCHUNK = EW // K
    RPT = N_PAD // NS
    mesh = plsc.VectorSubcoreMesh(core_axis_name="c", subcore_axis_name="s")

    NBUF = 10
    NSTEP = NCHUNK // NBUF
    TAIL = NCHUNK % NBUF

    @functools.partial(
        pl.kernel,
        mesh=mesh,
        compiler_params=pltpu.CompilerParams(use_tc_tiling_on_sc=False),
        out_type=jax.ShapeDtypeStruct((NC, N_PAD, D), jnp.float32),
        scratch_types=[
            pltpu.VMEM((NCHUNK, K), jnp.int32),
            pltpu.VMEM((NCHUNK, K), jnp.int32),
            [pltpu.VMEM((K, D), jnp.float32)] * NBUF,
            pltpu.VMEM_SHARED((N_PAD, D), jnp.float32),
            [pltpu.SemaphoreType.DMA] * NBUF,
            [pltpu.SemaphoreType.DMA] * NBUF,
        ],
    )
    def scatter_kernel(y_hbm, src_hbm, dst_hbm, out_hbm,
                       srcb, dstb, rows, acc, sem_g, sem_s):
        c = lax.axis_index("c")
        s = lax.axis_index("s")
        w = c * NS + s
        pltpu.async_copy(src_hbm.at[pl.ds(w * NCHUNK, NCHUNK)], srcb, sem_g[0])
        pltpu.async_copy(dst_hbm.at[pl.ds(w * NCHUNK, NCHUNK)], dstb, sem_g[1])

        def zstep(r, carry):
            for j in range(D // 16):
                rows[0][r, pl.ds(j * 16, 16)] = jnp.zeros((16,), jnp.float32)
            return carry

        lax.fori_loop(0, K, zstep, 0, unroll=False)
        for t in range((RPT + K - 1) // K):
            sz = min(K, RPT - t * K)
            pltpu.sync_copy(rows[0].at[pl.ds(0, sz)],
                            acc.at[pl.ds(s * RPT + t * K, sz)])
        pltpu.make_async_copy(src_hbm.at[pl.ds(w * NCHUNK, NCHUNK)], srcb,
                              sem_g[0]).wait()
        pltpu.make_async_copy(dst_hbm.at[pl.ds(w * NCHUNK, NCHUNK)], dstb,
                              sem_g[1]).wait()
        plsc.subcore_barrier()

        for b in range(NBUF):
            pltpu.async_copy(y_hbm.at[srcb.at[b]], rows[b], sem_g[b])

        def step(jj, carry):
            j0 = jj * NBUF
            for b in range(NBUF):
                pltpu.make_async_copy(y_hbm.at[srcb.at[j0 + b]],
                                      rows[b], sem_g[b]).wait()
                pltpu.async_copy(rows[b], acc.at[dstb.at[j0 + b]],
                                 sem_s[b], add=True)
            for b in range(NBUF):
                pltpu.make_async_copy(rows[b], acc.at[dstb.at[j0 + b]],
                                      sem_s[b]).wait()

                @pl.when(j0 + b + NBUF < NCHUNK)
                def _():
                    pltpu.async_copy(y_hbm.at[srcb.at[j0 + b + NBUF]],
                                     rows[b], sem_g[b])

            return carry

        lax.fori_loop(0, NSTEP, step, 0, unroll=False)
        base = NSTEP * NBUF
        for b in range(TAIL):
            pltpu.make_async_copy(y_hbm.at[srcb.at[base + b]],
                                  rows[b], sem_g[b]).wait()
            pltpu.async_copy(rows[b], acc.at[dstb.at[base + b]],
                             sem_s[b], add=True)
        for b in range(TAIL):
            pltpu.make_async_copy(rows[b], acc.at[dstb.at[base + b]],
                                  sem_s[b]).wait()
        plsc.subcore_barrier()
        nt = (RPT + K - 1) // K
        for t in range(nt):
            sz = min(K, RPT - t * K)
            b = t % NBUF
            pltpu.async_copy(acc.at[pl.ds(s * RPT + t * K, sz)],
                             rows[b].at[pl.ds(0, sz)], sem_g[b])
        for t in range(nt):
            sz = min(K, RPT - t * K)
            b = t % NBUF
            pltpu.make_async_copy(acc.at[pl.ds(s * RPT + t * K, sz)],
                                  rows[b].at[pl.ds(0, sz)], sem_g[b]).wait()
            pltpu.async_copy(rows[b].at[pl.ds(0, sz)],
                             out_hbm.at[c, pl.ds(s * RPT + t * K, sz)],
                             sem_s[b])
        for t in range(nt):
            sz = min(K, RPT - t * K)
            b = t % NBUF
            pltpu.make_async_copy(rows[b].at[pl.ds(0, sz)],
                                  out_hbm.at[c, pl.ds(s * RPT + t * K, sz)],
                                  sem_s[b]).wait()

    return scatter_kernel


def _dinvs(d0e_ref, d0o_ref, d1e_ref, d1o_ref):
    dE = lax.rsqrt(d0e_ref[...] + d1e_ref[...] + 1.0)
    dO = lax.rsqrt(d0o_ref[...] + d1o_ref[...] + 1.0)
    return dE, dO


def _tc1_body(x_ref, w_ref, d0e_ref, d0o_ref, d1e_ref, d1o_ref,
              y_ref, *, h):
    dE, dO = _dinvs(d0e_ref, d0o_ref, d1e_ref, d1o_ref)
    w = w_ref[...]
    pr = y_ref.shape[0]
    xp = x_ref[...].reshape(pr, 2, x_ref.shape[1])
    xwE = jnp.dot(xp[:, 0, :], w, preferred_element_type=jnp.float32)
    xwO = jnp.dot(xp[:, 1, :], w, preferred_element_type=jnp.float32)
    y_ref[...] = jnp.concatenate([xwE * dE, xwO * dO], axis=1)


def _tc2_body(p_ref, y_ref, d0e_ref, d0o_ref, d1e_ref, d1o_ref,
              b_ref, w_ref, out_ref, *, h):
    dE, dO = _dinvs(d0e_ref, d0o_ref, d1e_ref, d1o_ref)
    ssum = p_ref[0] + p_ref[1] + y_ref[...]
    b = b_ref[...]
    w = w_ref[...]
    hE = jnp.maximum(ssum[:, :h] * dE + b, 0.0)
    hO = jnp.maximum(ssum[:, h:] * dO + b, 0.0)
    yE = jnp.dot(hE, w, preferred_element_type=jnp.float32) * dE
    yO = jnp.dot(hO, w, preferred_element_type=jnp.float32) * dO
    out_ref[...] = jnp.concatenate([yE, yO], axis=1)


def _tc3_body(p_ref, y_ref, d0e_ref, d0o_ref, d1e_ref, d1o_ref,
              b_ref, wfc_ref, bfc_ref, out_ref, acc_ref,
              *, h, n_nodes, n_blocks):
    i = pl.program_id(0)
    dE, dO = _dinvs(d0e_ref, d0o_ref, d1e_ref, d1o_ref)
    ssum = p_ref[0] + p_ref[1] + y_ref[...]
    b = b_ref[...]
    hE = jnp.maximum(ssum[:, :h] * dE + b, 0.0)
    hO = jnp.maximum(ssum[:, h:] * dO + b, 0.0)
    colsum = jnp.sum(hE, axis=0, keepdims=True) + jnp.sum(
        hO, axis=0, keepdims=True)

    @pl.when(i == 0)
    def _():
        acc_ref[...] = colsum

    @pl.when(i > 0)
    def _():
        acc_ref[...] = acc_ref[...] + colsum

    @pl.when(i == n_blocks - 1)
    def _():
        mean = acc_ref[...] * (1.0 / n_nodes)
        out_ref[...] = (jnp.dot(mean, wfc_ref[...],
                                preferred_element_type=jnp.float32)
                        + bfc_ref[...])


def kernel(x, edge_index, W1, b1, W2, b2, Wfc, bfc):
    N, D_IN = x.shape
    E = edge_index.shape[1]
    H1 = W1.shape[1]
    H2 = W2.shape[1]
    D_OUT = Wfc.shape[1]

    NC, NS = 2, 16
    NW = NC * NS
    K = 80
    RPT_ALIGN = 8 * NS
    N_PAD = ((N + RPT_ALIGN - 1) // RPT_ALIGN) * RPT_ALIGN

    ei_flat = edge_index.reshape(2 * E)
    src2 = ei_flat[:E].reshape(E // K, K)
    dst2 = ei_flat[E:].reshape(E // K, K)

    deg_k = _make_deg_kernel(E, N_PAD, K, NC, NS)
    scat_k = _make_scatter_kernel(N, E, N_PAD, H1, K, NC, NS)

    degp = deg_k(dst2).reshape(NC, N_PAD)
    NP2 = N // 2
    N_PAD2 = N_PAD // 2
    DP = 2 * H1
    d0e = degp[0, 0::2].reshape(N_PAD2, 1)
    d0o = degp[0, 1::2].reshape(N_PAD2, 1)
    d1e = degp[1, 0::2].reshape(N_PAD2, 1)
    d1o = degp[1, 1::2].reshape(N_PAD2, 1)

    PR = 1000
    n_blocks = NP2 // PR
    deg_spec = pl.BlockSpec((PR, 1), lambda i: (i, 0))
    part_spec = pl.BlockSpec((NC, PR, DP), lambda i: (0, i, 0))
    pair_spec = pl.BlockSpec((PR, DP), lambda i: (i, 0))

    y1p = pl.pallas_call(
        functools.partial(_tc1_body, h=H1),
        grid=(n_blocks,),
        in_specs=[
            pl.BlockSpec((2 * PR, D_IN), lambda i: (i, 0)),
            pl.BlockSpec((D_IN, H1), lambda i: (0, 0)),
            deg_spec,
            deg_spec,
            deg_spec,
            deg_spec,
        ],
        out_specs=pair_spec,
        out_shape=jax.ShapeDtypeStruct((NP2, DP), jnp.float32),
    )(x, W1, d0e, d0o, d1e, d1o)

    p1 = scat_k(y1p.reshape(N, H1), src2, dst2)
    p1p = p1.reshape(NC, N_PAD2, DP)

    y2p = pl.pallas_call(
        functools.partial(_tc2_body, h=H1),
        grid=(n_blocks,),
        in_specs=[
            part_spec,
            pair_spec,
            deg_spec,
            deg_spec,
            deg_spec,
            deg_spec,
            pl.BlockSpec((1, H1), lambda i: (0, 0)),
            pl.BlockSpec((H1, H2), lambda i: (0, 0)),
        ],
        out_specs=pair_spec,
        out_shape=jax.ShapeDtypeStruct((NP2, DP), jnp.float32),
    )(p1p, y1p, d0e, d0o, d1e, d1o, b1.reshape(1, H1), W2)

    p2 = scat_k(y2p.reshape(N, H2), src2, dst2)
    p2p = p2.reshape(NC, N_PAD2, DP)

    out = pl.pallas_call(
        functools.partial(_tc3_body, h=H2, n_nodes=N, n_blocks=n_blocks),
        grid=(n_blocks,),
        in_specs=[
            part_spec,
            pair_spec,
            deg_spec,
            deg_spec,
            deg_spec,
            deg_spec,
            pl.BlockSpec((1, H2), lambda i: (0, 0)),
            pl.BlockSpec((H2, D_OUT), lambda i: (0, 0)),
            pl.BlockSpec((1, D_OUT), lambda i: (0, 0)),
        ],
        out_specs=pl.BlockSpec((1, D_OUT), lambda i: (0, 0)),
        out_shape=jax.ShapeDtypeStruct((1, D_OUT), jnp.float32),
        scratch_shapes=[pltpu.VMEM((1, D_OUT), jnp.float32)],
    )(p2p, y2p, d0e, d0o, d1e, d1o, b2.reshape(1, H2), Wfc,
      bfc.reshape(1, D_OUT))

    return out


# speedup vs baseline: 1.0060x; 1.0060x over previous
import functools

import jax
import jax.numpy as jnp
from jax import lax
from jax.experimental import pallas as pl
from jax.experimental.pallas import tpu as pltpu
from jax.experimental.pallas import tpu_sc as plsc


def _make_deg_kernel(E, N_PAD, K, NC, NS):
    EW = E // (NC * NS)
    NCHUNK = EW // K
    RPT = N_PAD // NS
    mesh = plsc.VectorSubcoreMesh(core_axis_name="c", subcore_axis_name="s")

    BUF = ((RPT + 15) // 16) * 16
    GRP = 5
    assert NCHUNK % GRP == 0

    @functools.partial(
        pl.kernel,
        mesh=mesh,
        compiler_params=pltpu.CompilerParams(use_tc_tiling_on_sc=False),
        out_type=jax.ShapeDtypeStruct((NC * N_PAD,), jnp.float32),
        scratch_types=[
            pltpu.VMEM((NCHUNK, K), jnp.int32),
            pltpu.VMEM((K,), jnp.float32),
            pltpu.VMEM((BUF,), jnp.float32),
            pltpu.VMEM_SHARED((N_PAD,), jnp.float32),
            pltpu.SemaphoreType.DMA,
        ],
    )
    def deg_kernel(dst_hbm, out_hbm, dstb, onesv, buf, acc, sem):
        c = lax.axis_index("c")
        s = lax.axis_index("s")
        w = c * NS + s
        for i in range(K // 16):
            onesv[pl.ds(i * 16, 16)] = jnp.ones((16,), jnp.float32)

        def zstep(i, carry):
            buf[pl.ds(i * 16, 16)] = jnp.zeros((16,), jnp.float32)
            return carry

        lax.fori_loop(0, BUF // 16, zstep, 0, unroll=False)
        pltpu.sync_copy(dst_hbm.at[pl.ds(w * NCHUNK, NCHUNK)], dstb)
        pltpu.sync_copy(buf.at[pl.ds(0, RPT)], acc.at[pl.ds(s * RPT, RPT)])
        plsc.subcore_barrier()

        def step(g, carry):
            for i in range(GRP):
                pltpu.async_copy(onesv, acc.at[dstb.at[g * GRP + i]], sem,
                                 add=True)
            for i in range(GRP):
                pltpu.make_async_copy(onesv, acc.at[dstb.at[g * GRP + i]],
                                      sem).wait()
            return carry

        lax.fori_loop(0, NCHUNK // GRP, step, 0, unroll=False)
        plsc.subcore_barrier()
        pltpu.sync_copy(acc.at[pl.ds(s * RPT, RPT)], buf.at[pl.ds(0, RPT)])
        pltpu.sync_copy(buf.at[pl.ds(0, RPT)],
                        out_hbm.at[pl.ds(c * N_PAD + s * RPT, RPT)])

    return deg_kernel


def _make_scatter_kernel(N, E, N_PAD, D, K, NC, NS):
    EW = E // (NC * NS)
    NCHUNK = EW // K
    RPT = N_PAD // NS
    mesh = plsc.VectorSubcoreMesh(core_axis_name="c", subcore_axis_name="s")

    NBUF = 10
    NSTEP = NCHUNK // NBUF
    TAIL = NCHUNK % NBUF

    @functools.partial(
        pl.kernel,
        mesh=mesh,
        compiler_params=pltpu.CompilerParams(use_tc_tiling_on_sc=False),
        out_type=jax.ShapeDtypeStruct((NC, N_PAD, D), jnp.float32),
        scratch_types=[
            pltpu.VMEM((NCHUNK, K), jnp.int32),
            pltpu.VMEM((NCHUNK, K), jnp.int32),
            [pltpu.VMEM((K, D), jnp.float32)] * NBUF,
            pltpu.VMEM_SHARED((N_PAD, D), jnp.float32),
            [pltpu.SemaphoreType.DMA] * NBUF,
            [pltpu.SemaphoreType.DMA] * NBUF,
        ],
    )
    def scatter_kernel(y_hbm, src_hbm, dst_hbm, out_hbm,
                       srcb, dstb, rows, acc, sem_g, sem_s):
        c = lax.axis_index("c")
        s = lax.axis_index("s")
        w = c * NS + s
        pltpu.async_copy(src_hbm.at[pl.ds(w * NCHUNK, NCHUNK)], srcb, sem_g[0])
        pltpu.async_copy(dst_hbm.at[pl.ds(w * NCHUNK, NCHUNK)], dstb, sem_g[1])

        def zstep(r, carry):
            for j in range(D // 16):
                rows[0][r, pl.ds(j * 16, 16)] = jnp.zeros((16,), jnp.float32)
            return carry

        lax.fori_loop(0, K, zstep, 0, unroll=False)
        for t in range((RPT + K - 1) // K):
            sz = min(K, RPT - t * K)
            pltpu.sync_copy(rows[0].at[pl.ds(0, sz)],
                            acc.at[pl.ds(s * RPT + t * K, sz)])
        pltpu.make_async_copy(src_hbm.at[pl.ds(w * NCHUNK, NCHUNK)], srcb,
                              sem_g[0]).wait()
        pltpu.make_async_copy(dst_hbm.at[pl.ds(w * NCHUNK, NCHUNK)], dstb,
                              sem_g[1]).wait()
        plsc.subcore_barrier()

        for b in range(NBUF):
            pltpu.async_copy(y_hbm.at[srcb.at[b]], rows[b], sem_g[b])

        def step(jj, carry):
            j0 = jj * NBUF
            for b in range(NBUF):
                pltpu.make_async_copy(y_hbm.at[srcb.at[j0 + b]],
                                      rows[b], sem_g[b]).wait()
                pltpu.async_copy(rows[b], acc.at[dstb.at[j0 + b]],
                                 sem_s[b], add=True)
            for b in range(NBUF):
                pltpu.make_async_copy(rows[b], acc.at[dstb.at[j0 + b]],
                                      sem_s[b]).wait()

                @pl.when(j0 + b + NBUF < NCHUNK)
                def _():
                    pltpu.async_copy(y_hbm.at[srcb.at[j0 + b + NBUF]],
                                     rows[b], sem_g[b])

            return carry

        lax.fori_loop(0, NSTEP, step, 0, unroll=False)
        base = NSTEP * NBUF
        for b in range(TAIL):
            pltpu.make_async_copy(y_hbm.at[srcb.at[base + b]],
                                  rows[b], sem_g[b]).wait()
            pltpu.async_copy(rows[b], acc.at[dstb.at[base + b]],
                             sem_s[b], add=True)
        for b in range(TAIL):
            pltpu.make_async_copy(rows[b], acc.at[dstb.at[base + b]],
                                  sem_s[b]).wait()
        plsc.subcore_barrier()
        nt = (RPT + K - 1) // K
        for t in range(nt):
            sz = min(K, RPT - t * K)
            b = t % NBUF
            pltpu.async_copy(acc.at[pl.ds(s * RPT + t * K, sz)],
                             rows[b].at[pl.ds(0, sz)], sem_g[b])
        for t in range(nt):
            sz = min(K, RPT - t * K)
            b = t % NBUF
            pltpu.make_async_copy(acc.at[pl.ds(s * RPT + t * K, sz)],
                                  rows[b].at[pl.ds(0, sz)], sem_g[b]).wait()
            pltpu.async_copy(rows[b].at[pl.ds(0, sz)],
                             out_hbm.at[c, pl.ds(s * RPT + t * K, sz)],
                             sem_s[b])
        for t in range(nt):
            sz = min(K, RPT - t * K)
            b = t % NBUF
            pltpu.make_async_copy(rows[b].at[pl.ds(0, sz)],
                                  out_hbm.at[c, pl.ds(s * RPT + t * K, sz)],
                                  sem_s[b]).wait()

    return scatter_kernel


def _dinvs(d0e_ref, d0o_ref, d1e_ref, d1o_ref):
    dE = lax.rsqrt(d0e_ref[...] + d1e_ref[...] + 1.0)
    dO = lax.rsqrt(d0o_ref[...] + d1o_ref[...] + 1.0)
    return dE, dO


def _tc1_body(x_ref, w_ref, d0e_ref, d0o_ref, d1e_ref, d1o_ref,
              y_ref, *, h):
    dE, dO = _dinvs(d0e_ref, d0o_ref, d1e_ref, d1o_ref)
    w = w_ref[...]
    pr = y_ref.shape[0]
    xp = x_ref[...].reshape(pr, 2, x_ref.shape[1])
    xwE = jnp.dot(xp[:, 0, :], w, preferred_element_type=jnp.float32)
    xwO = jnp.dot(xp[:, 1, :], w, preferred_element_type=jnp.float32)
    y_ref[...] = jnp.concatenate([xwE * dE, xwO * dO], axis=1)


def _tc2_body(p_ref, y_ref, d0e_ref, d0o_ref, d1e_ref, d1o_ref,
              b_ref, w_ref, out_ref, *, h):
    dE, dO = _dinvs(d0e_ref, d0o_ref, d1e_ref, d1o_ref)
    ssum = p_ref[0] + p_ref[1] + y_ref[...]
    b = b_ref[...]
    w = w_ref[...]
    hE = jnp.maximum(ssum[:, :h] * dE + b, 0.0)
    hO = jnp.maximum(ssum[:, h:] * dO + b, 0.0)
    yE = jnp.dot(hE, w, preferred_element_type=jnp.float32) * dE
    yO = jnp.dot(hO, w, preferred_element_type=jnp.float32) * dO
    out_ref[...] = jnp.concatenate([yE, yO], axis=1)


def _tc3_body(p_ref, y_ref, d0e_ref, d0o_ref, d1e_ref, d1o_ref,
              b_ref, wfc_ref, bfc_ref, out_ref, acc_ref,
              *, h, n_nodes, n_blocks):
    i = pl.program_id(0)
    dE, dO = _dinvs(d0e_ref, d0o_ref, d1e_ref, d1o_ref)
    ssum = p_ref[0] + p_ref[1] + y_ref[...]
    b = b_ref[...]
    hE = jnp.maximum(ssum[:, :h] * dE + b, 0.0)
    hO = jnp.maximum(ssum[:, h:] * dO + b, 0.0)
    colsum = jnp.sum(hE, axis=0, keepdims=True) + jnp.sum(
        hO, axis=0, keepdims=True)

    @pl.when(i == 0)
    def _():
        acc_ref[...] = colsum

    @pl.when(i > 0)
    def _():
        acc_ref[...] = acc_ref[...] + colsum

    @pl.when(i == n_blocks - 1)
    def _():
        mean = acc_ref[...] * (1.0 / n_nodes)
        out_ref[...] = (jnp.dot(mean, wfc_ref[...],
                                preferred_element_type=jnp.float32)
                        + bfc_ref[...])


def kernel(x, edge_index, W1, b1, W2, b2, Wfc, bfc):
    N, D_IN = x.shape
    E = edge_index.shape[1]
    H1 = W1.shape[1]
    H2 = W2.shape[1]
    D_OUT = Wfc.shape[1]

    NC, NS = 2, 16
    NW = NC * NS
    K = 80
    RPT_ALIGN = 8 * NS
    N_PAD = ((N + RPT_ALIGN - 1) // RPT_ALIGN) * RPT_ALIGN

    src2 = edge_index[0].reshape(E // K, K)
    dst2 = edge_index[1].reshape(E // K, K)

    deg_k = _make_deg_kernel(E, N_PAD, K, NC, NS)
    scat_k = _make_scatter_kernel(N, E, N_PAD, H1, K, NC, NS)

    degp = deg_k(dst2).reshape(NC, N_PAD)
    NP2 = N // 2
    N_PAD2 = N_PAD // 2
    DP = 2 * H1
    d0e = degp[0, 0::2].reshape(N_PAD2, 1)
    d0o = degp[0, 1::2].reshape(N_PAD2, 1)
    d1e = degp[1, 0::2].reshape(N_PAD2, 1)
    d1o = degp[1, 1::2].reshape(N_PAD2, 1)

    PR = 1000
    n_blocks = NP2 // PR
    deg_spec = pl.BlockSpec((PR, 1), lambda i: (i, 0))
    part_spec = pl.BlockSpec((NC, PR, DP), lambda i: (0, i, 0))
    pair_spec = pl.BlockSpec((PR, DP), lambda i: (i, 0))

    y1p = pl.pallas_call(
        functools.partial(_tc1_body, h=H1),
        grid=(n_blocks,),
        in_specs=[
            pl.BlockSpec((2 * PR, D_IN), lambda i: (i, 0)),
            pl.BlockSpec((D_IN, H1), lambda i: (0, 0)),
            deg_spec,
            deg_spec,
            deg_spec,
            deg_spec,
        ],
        out_specs=pair_spec,
        out_shape=jax.ShapeDtypeStruct((NP2, DP), jnp.float32),
    )(x, W1, d0e, d0o, d1e, d1o)

    p1 = scat_k(y1p.reshape(N, H1), src2, dst2)
    p1p = p1.reshape(NC, N_PAD2, DP)

    y2p = pl.pallas_call(
        functools.partial(_tc2_body, h=H1),
        grid=(n_blocks,),
        in_specs=[
            part_spec,
            pair_spec,
            deg_spec,
            deg_spec,
            deg_spec,
            deg_spec,
            pl.BlockSpec((1, H1), lambda i: (0, 0)),
            pl.BlockSpec((H1, H2), lambda i: (0, 0)),
        ],
        out_specs=pair_spec,
        out_shape=jax.ShapeDtypeStruct((NP2, DP), jnp.float32),
    )(p1p, y1p, d0e, d0o, d1e, d1o, b1.reshape(1, H1), W2)

    p2 = scat_k(y2p.reshape(N, H2), src2, dst2)
    p2p = p2.reshape(NC, N_PAD2, DP)

    out = pl.pallas_call(
        functools.partial(_tc3_body, h=H2, n_nodes=N, n_blocks=n_blocks),
        grid=(n_blocks,),
        in_specs=[
            part_spec,
            pair_spec,
            deg_spec,
            deg_spec,
            deg_spec,
            deg_spec,
            pl.BlockSpec((1, H2), lambda i: (0, 0)),
            pl.BlockSpec((H2, D_OUT), lambda i: (0, 0)),
            pl.BlockSpec((1, D_OUT), lambda i: (0, 0)),
        ],
        out_specs=pl.BlockSpec((1, D_OUT), lambda i: (0, 0)),
        out_shape=jax.ShapeDtypeStruct((1, D_OUT), jnp.float32),
        scratch_shapes=[pltpu.VMEM((1, D_OUT), jnp.float32)],
    )(p2p, y2p, d0e, d0o, d1e, d1o, b2.reshape(1, H2), Wfc,
      bfc.reshape(1, D_OUT))

    return out
